# Initial kernel scaffold; baseline (speedup 1.0000x reference)
import functools
import math

import jax
import jax.numpy as jnp
import numpy as np
from jax.experimental import pallas as pl
from jax.experimental.pallas import tpu as pltpu

N_NODE = 50000
C = 128
H = 4
DH = 32
BN = 2000


def _proj_relu_body(x_ref, w_ref, b_ref, o_ref):
    y = jnp.dot(x_ref[...], w_ref[...], preferred_element_type=jnp.float32)
    o_ref[...] = jax.nn.relu(y + b_ref[...])


def _proj_relu(x, w, b):
    n = x.shape[0]
    grid = (n // BN,)
    return pl.pallas_call(
        _proj_relu_body,
        grid=grid,
        in_specs=[
            pl.BlockSpec((BN, x.shape[1]), lambda i: (i, 0)),
            pl.BlockSpec((x.shape[1], w.shape[1]), lambda i: (0, 0)),
            pl.BlockSpec((1, w.shape[1]), lambda i: (0, 0)),
        ],
        out_specs=pl.BlockSpec((BN, w.shape[1]), lambda i: (i, 0)),
        out_shape=jax.ShapeDtypeStruct((n, w.shape[1]), jnp.float32),
    )(x, w, b.reshape(1, -1))


def _proj_body(x_ref, w_ref, o_ref):
    o_ref[...] = jnp.dot(x_ref[...], w_ref[...], preferred_element_type=jnp.float32)


def _proj(x, w):
    n = x.shape[0]
    grid = (n // BN,)
    return pl.pallas_call(
        _proj_body,
        grid=grid,
        in_specs=[
            pl.BlockSpec((BN, x.shape[1]), lambda i: (i, 0)),
            pl.BlockSpec((x.shape[1], w.shape[1]), lambda i: (0, 0)),
        ],
        out_specs=pl.BlockSpec((BN, w.shape[1]), lambda i: (i, 0)),
        out_shape=jax.ShapeDtypeStruct((n, w.shape[1]), jnp.float32),
    )(x, w)


def _out_stage_body(m_ref, h_ref, w_ref, b_ref, o_ref, *, beta, hcoef):
    o = jax.nn.gelu(m_ref[...])
    o = jnp.dot(o, w_ref[...], preferred_element_type=jnp.float32) + b_ref[...]
    o_ref[...] = beta * o + hcoef * h_ref[...]


def _out_stage(m, h, w, b, beta, hcoef):
    n = m.shape[0]
    grid = (n // BN,)
    return pl.pallas_call(
        functools.partial(_out_stage_body, beta=beta, hcoef=hcoef),
        grid=grid,
        in_specs=[
            pl.BlockSpec((BN, C), lambda i: (i, 0)),
            pl.BlockSpec((BN, C), lambda i: (i, 0)),
            pl.BlockSpec((C, C), lambda i: (0, 0)),
            pl.BlockSpec((1, C), lambda i: (0, 0)),
        ],
        out_specs=pl.BlockSpec((BN, C), lambda i: (i, 0)),
        out_shape=jax.ShapeDtypeStruct((n, C), jnp.float32),
    )(m, h, w, b.reshape(1, -1))


def _edge_phase(q_dst, krel_src, vrel_src, s, d, n_dst):
    ke = jnp.take(krel_src, s, axis=0).reshape(-1, H, DH)
    qe = jnp.take(q_dst, d, axis=0).reshape(-1, H, DH)
    alpha = (qe * ke).sum(-1)
    ex = jnp.exp(alpha)
    denom = jax.ops.segment_sum(ex, d, num_segments=n_dst)
    a = ex / (jnp.take(denom, d, axis=0) + 1e-16)
    ve = jnp.take(vrel_src, s, axis=0).reshape(-1, H, DH)
    msg = (ve * a[:, :, None]).reshape(-1, C)
    return jax.ops.segment_sum(msg, d, num_segments=n_dst)


_SRC_EDGE = {'user': 'ui', 'item': 'iu'}
_EDGE_DEFS = (('ui', 'user', 'item'), ('iu', 'item', 'user'))


def _fold_params(params):
    folded = {}
    inv_sqrt = 1.0 / math.sqrt(float(DH))
    for l in range(2):
        for t in ('user', 'item'):
            e = _SRC_EDGE[t]
            arel = params['l%d_arel_%s' % (l, e)]
            mrel = params['l%d_mrel_%s' % (l, e)]
            prel = params['l%d_prel_%s' % (l, e)] * inv_sqrt
            Wk = params['l%d_Wk_%s' % (l, t)].reshape(C, H, DH)
            Wv = params['l%d_Wv_%s' % (l, t)].reshape(C, H, DH)
            Wk_f = jnp.einsum('chd,hde,h->che', Wk, arel, prel).reshape(C, C)
            Wv_f = jnp.einsum('chd,hde->che', Wv, mrel).reshape(C, C)
            Wq = params['l%d_Wq_%s' % (l, t)]
            folded['Wqkv_%d_%s' % (l, t)] = jnp.concatenate([Wq, Wk_f, Wv_f], axis=1)
    return folded


def kernel(x_user, x_item, edge_index_user_item, edge_index_item_user, params):
    folded = _fold_params(params)
    h = {'user': _proj_relu(x_user, params['in_W_user'], params['in_b_user']),
         'item': _proj_relu(x_item, params['in_W_item'], params['in_b_item'])}
    ei = {'ui': (edge_index_user_item[0], edge_index_user_item[1]),
          'iu': (edge_index_item_user[0], edge_index_item_user[1])}
    for l in range(2):
        q, krel, vrel = {}, {}, {}
        for t in h:
            y = _proj(h[t], folded['Wqkv_%d_%s' % (l, t)])
            q[t] = y[:, :C]
            krel[t] = y[:, C:2 * C]
            vrel[t] = y[:, 2 * C:]
        out = {}
        for e, src, dst in _EDGE_DEFS:
            s, d = ei[e]
            out[dst] = _edge_phase(q[dst], krel[src], vrel[src], s, d, N_NODE)
        h_new = {}
        for t in h:
            beta = jax.nn.sigmoid(params['l%d_skip_%s' % (l, t)])
            hcoef = (1.0 - beta) + (1.0 if l > 0 else 0.0)
            h_new[t] = _out_stage(out[t], h[t],
                                  params['l%d_Wa_%s' % (l, t)],
                                  params['l%d_ba_%s' % (l, t)], beta, hcoef)
        h = h_new
    return (h['user'], h['item'])


# speedup vs baseline: 10.0531x; 10.0531x over previous
import functools
import math

import jax
import jax.numpy as jnp
import numpy as np
from jax.experimental import pallas as pl
from jax.experimental.pallas import tpu as pltpu

N_NODE = 50000
C = 128
H = 4
DH = 32
BN = 2000


def _proj_relu_body(x_ref, w_ref, b_ref, o_ref):
    y = jnp.dot(x_ref[...], w_ref[...], preferred_element_type=jnp.float32)
    o_ref[...] = jax.nn.relu(y + b_ref[...])


def _proj_relu(x, w, b):
    n = x.shape[0]
    grid = (n // BN,)
    return pl.pallas_call(
        _proj_relu_body,
        grid=grid,
        in_specs=[
            pl.BlockSpec((BN, x.shape[1]), lambda i: (i, 0)),
            pl.BlockSpec((x.shape[1], w.shape[1]), lambda i: (0, 0)),
            pl.BlockSpec((1, w.shape[1]), lambda i: (0, 0)),
        ],
        out_specs=pl.BlockSpec((BN, w.shape[1]), lambda i: (i, 0)),
        out_shape=jax.ShapeDtypeStruct((n, w.shape[1]), jnp.float32),
    )(x, w, b.reshape(1, -1))


def _proj_body(x_ref, w_ref, o_ref):
    o_ref[...] = jnp.dot(x_ref[...], w_ref[...], preferred_element_type=jnp.float32)


def _proj(x, w):
    n = x.shape[0]
    grid = (n // BN,)
    return pl.pallas_call(
        _proj_body,
        grid=grid,
        in_specs=[
            pl.BlockSpec((BN, x.shape[1]), lambda i: (i, 0)),
            pl.BlockSpec((x.shape[1], w.shape[1]), lambda i: (0, 0)),
        ],
        out_specs=pl.BlockSpec((BN, w.shape[1]), lambda i: (i, 0)),
        out_shape=jax.ShapeDtypeStruct((n, w.shape[1]), jnp.float32),
    )(x, w)


def _out_stage_body(c_ref, m_ref, h_ref, w_ref, b_ref, o_ref):
    o = jax.nn.gelu(m_ref[...])
    o = jnp.dot(o, w_ref[...], preferred_element_type=jnp.float32) + b_ref[...]
    o_ref[...] = c_ref[0] * o + c_ref[1] * h_ref[...]


def _out_stage(m, h, w, b, beta, hcoef):
    n = m.shape[0]
    grid = (n // BN,)
    coef = jnp.stack([beta, hcoef]).astype(jnp.float32)
    return pl.pallas_call(
        _out_stage_body,
        grid=grid,
        in_specs=[
            pl.BlockSpec(memory_space=pltpu.SMEM),
            pl.BlockSpec((BN, C), lambda i: (i, 0)),
            pl.BlockSpec((BN, C), lambda i: (i, 0)),
            pl.BlockSpec((C, C), lambda i: (0, 0)),
            pl.BlockSpec((1, C), lambda i: (0, 0)),
        ],
        out_specs=pl.BlockSpec((BN, C), lambda i: (i, 0)),
        out_shape=jax.ShapeDtypeStruct((n, C), jnp.float32),
    )(coef, m, h, w, b.reshape(1, -1))


def _edge_phase(q_dst, krel_src, vrel_src, s, d, n_dst):
    ke = jnp.take(krel_src, s, axis=0).reshape(-1, H, DH)
    qe = jnp.take(q_dst, d, axis=0).reshape(-1, H, DH)
    alpha = (qe * ke).sum(-1)
    ex = jnp.exp(alpha)
    denom = jax.ops.segment_sum(ex, d, num_segments=n_dst)
    a = ex / (jnp.take(denom, d, axis=0) + 1e-16)
    ve = jnp.take(vrel_src, s, axis=0).reshape(-1, H, DH)
    msg = (ve * a[:, :, None]).reshape(-1, C)
    return jax.ops.segment_sum(msg, d, num_segments=n_dst)


_SRC_EDGE = {'user': 'ui', 'item': 'iu'}
_EDGE_DEFS = (('ui', 'user', 'item'), ('iu', 'item', 'user'))


def _fold_params(params):
    folded = {}
    inv_sqrt = 1.0 / math.sqrt(float(DH))
    for l in range(2):
        for t in ('user', 'item'):
            e = _SRC_EDGE[t]
            arel = params['l%d_arel_%s' % (l, e)]
            mrel = params['l%d_mrel_%s' % (l, e)]
            prel = params['l%d_prel_%s' % (l, e)] * inv_sqrt
            Wk = params['l%d_Wk_%s' % (l, t)].reshape(C, H, DH)
            Wv = params['l%d_Wv_%s' % (l, t)].reshape(C, H, DH)
            Wk_f = jnp.einsum('chd,hde,h->che', Wk, arel, prel).reshape(C, C)
            Wv_f = jnp.einsum('chd,hde->che', Wv, mrel).reshape(C, C)
            Wq = params['l%d_Wq_%s' % (l, t)]
            folded['Wqkv_%d_%s' % (l, t)] = jnp.concatenate([Wq, Wk_f, Wv_f], axis=1)
    return folded


def kernel(x_user, x_item, edge_index_user_item, edge_index_item_user, params):
    folded = _fold_params(params)
    h = {'user': _proj_relu(x_user, params['in_W_user'], params['in_b_user']),
         'item': _proj_relu(x_item, params['in_W_item'], params['in_b_item'])}
    ei = {'ui': (edge_index_user_item[0], edge_index_user_item[1]),
          'iu': (edge_index_item_user[0], edge_index_item_user[1])}
    for l in range(2):
        q, krel, vrel = {}, {}, {}
        for t in h:
            y = _proj(h[t], folded['Wqkv_%d_%s' % (l, t)])
            q[t] = y[:, :C]
            krel[t] = y[:, C:2 * C]
            vrel[t] = y[:, 2 * C:]
        out = {}
        for e, src, dst in _EDGE_DEFS:
            s, d = ei[e]
            out[dst] = _edge_phase(q[dst], krel[src], vrel[src], s, d, N_NODE)
        h_new = {}
        for t in h:
            beta = jax.nn.sigmoid(params['l%d_skip_%s' % (l, t)])
            hcoef = (1.0 - beta) + (1.0 if l > 0 else 0.0)
            h_new[t] = _out_stage(out[t], h[t],
                                  params['l%d_Wa_%s' % (l, t)],
                                  params['l%d_ba_%s' % (l, t)], beta, hcoef)
        h = h_new
    return (h['user'], h['item'])


# speedup vs baseline: 15.9858x; 1.5901x over previous
import functools
import math

import jax
import jax.numpy as jnp
import numpy as np
from jax import lax
from jax.experimental import pallas as pl
from jax.experimental.pallas import tpu as pltpu
from jax.experimental.pallas import tpu_sc as plsc

N_NODE = 50000
C = 128
H = 4
DH = 32
BN = 2000

NC = 2
NS = 16
NW = NC * NS
CH = 128
NCHUNK = 74
EPT = NCHUNK * CH
E_PAD = NW * EPT
E_REAL = 300000
NSP = 50048
RPT = NSP // NS


def _proj_relu_body(x_ref, w_ref, b_ref, o_ref):
    y = jnp.dot(x_ref[...], w_ref[...], preferred_element_type=jnp.float32)
    o_ref[...] = jax.nn.relu(y + b_ref[...])


def _proj_relu(x, w, b):
    n = x.shape[0]
    grid = (n // BN,)
    return pl.pallas_call(
        _proj_relu_body,
        grid=grid,
        in_specs=[
            pl.BlockSpec((BN, x.shape[1]), lambda i: (i, 0)),
            pl.BlockSpec((x.shape[1], w.shape[1]), lambda i: (0, 0)),
            pl.BlockSpec((1, w.shape[1]), lambda i: (0, 0)),
        ],
        out_specs=pl.BlockSpec((BN, w.shape[1]), lambda i: (i, 0)),
        out_shape=jax.ShapeDtypeStruct((n, w.shape[1]), jnp.float32),
    )(x, w, b.reshape(1, -1))


def _proj_body(x_ref, w_ref, o_ref):
    o_ref[...] = jnp.dot(x_ref[...], w_ref[...], preferred_element_type=jnp.float32)


def _proj(x, w):
    n = x.shape[0]
    grid = (n // BN,)
    return pl.pallas_call(
        _proj_body,
        grid=grid,
        in_specs=[
            pl.BlockSpec((BN, x.shape[1]), lambda i: (i, 0)),
            pl.BlockSpec((x.shape[1], w.shape[1]), lambda i: (0, 0)),
        ],
        out_specs=pl.BlockSpec((BN, w.shape[1]), lambda i: (i, 0)),
        out_shape=jax.ShapeDtypeStruct((n, w.shape[1]), jnp.float32),
    )(x, w)


def _out_stage_body(c_ref, m_ref, h_ref, w_ref, b_ref, o_ref):
    o = jax.nn.gelu(m_ref[...])
    o = jnp.dot(o, w_ref[...], preferred_element_type=jnp.float32) + b_ref[...]
    o_ref[...] = c_ref[0] * o + c_ref[1] * h_ref[...]


def _out_stage(m, h, w, b, beta, hcoef):
    n = m.shape[0]
    grid = (n // BN,)
    coef = jnp.stack([beta, hcoef]).astype(jnp.float32)
    return pl.pallas_call(
        _out_stage_body,
        grid=grid,
        in_specs=[
            pl.BlockSpec(memory_space=pltpu.SMEM),
            pl.BlockSpec((BN, C), lambda i: (i, 0)),
            pl.BlockSpec((BN, C), lambda i: (i, 0)),
            pl.BlockSpec((C, C), lambda i: (0, 0)),
            pl.BlockSpec((1, C), lambda i: (0, 0)),
        ],
        out_specs=pl.BlockSpec((BN, C), lambda i: (i, 0)),
        out_shape=jax.ShapeDtypeStruct((n, C), jnp.float32),
    )(coef, m, h, w, b.reshape(1, -1))


def _pass1_body(q_hbm, krel_hbm, sidx_hbm, didx_hbm,
                ex_hbm, den0_hbm, den1_hbm,
                sidx_v, didx_v, ke_v, qe_v, ex_v, den_sp, sem1, sem2):
    cid = lax.axis_index("c")
    sid = lax.axis_index("s")
    wid = cid * NS + sid
    iot = lax.iota(jnp.int32, 16)
    zero16 = jnp.zeros((16,), jnp.float32)

    for r in range(CH):
        ex_v[r] = zero16
    row0 = sid * RPT
    for j in range(RPT // CH):
        pltpu.sync_copy(ex_v, den_sp.at[pl.ds(row0 + j * CH, CH), :])
    rem = RPT - (RPT // CH) * CH
    if rem:
        pltpu.sync_copy(ex_v.at[pl.ds(0, rem), :],
                        den_sp.at[pl.ds(row0 + (RPT // CH) * CH, rem), :])
    plsc.subcore_barrier()

    def chunk_body(c, carry):
        base = (wid * NCHUNK + c) * CH
        pltpu.sync_copy(sidx_hbm.at[pl.ds(base, CH)], sidx_v)
        pltpu.sync_copy(didx_hbm.at[pl.ds(base, CH)], didx_v)
        cp1 = pltpu.async_copy(krel_hbm.at[sidx_v], ke_v, sem1)
        cp2 = pltpu.async_copy(q_hbm.at[didx_v], qe_v, sem2)
        cp1.wait()
        cp2.wait()
        for g in range(CH // 16):
            rows = iot + g * 16
            for h in range(H):
                def dd_body(dd, acc):
                    colv = jnp.full((16,), h * DH + dd, jnp.int32)
                    kv = plsc.load_gather(ke_v, [rows, colv])
                    qv = plsc.load_gather(qe_v, [rows, colv])
                    return acc + kv * qv
                acc_h = lax.fori_loop(0, DH, dd_body, zero16, unroll=8)
                mask = (base + g * 16 + iot) < E_REAL
                exh = jnp.where(mask, jnp.exp(acc_h), 0.0)
                plsc.store_scatter(ex_v, [rows, jnp.full((16,), h, jnp.int32)], exh)
        pltpu.sync_copy(ex_v, ex_hbm.at[pl.ds(base, CH), :])
        pltpu.sync_copy(ex_v, den_sp.at[didx_v], add=True)
        return carry

    lax.fori_loop(0, NCHUNK, chunk_body, 0)
    plsc.subcore_barrier()

    @pl.when(cid == 0)
    def _():
        pltpu.sync_copy(den_sp.at[pl.ds(row0, RPT), :],
                        den0_hbm.at[pl.ds(row0, RPT), :])

    @pl.when(cid == 1)
    def _():
        pltpu.sync_copy(den_sp.at[pl.ds(row0, RPT), :],
                        den1_hbm.at[pl.ds(row0, RPT), :])


def _sc_pass1(q_dst, krel_src, s_pad, d_pad):
    mesh = plsc.VectorSubcoreMesh(core_axis_name="c", subcore_axis_name="s",
                                  num_cores=NC, num_subcores=NS)
    fn = pl.kernel(
        _pass1_body,
        out_type=[
            jax.ShapeDtypeStruct((E_PAD, 16), jnp.float32),
            jax.ShapeDtypeStruct((NSP, 16), jnp.float32),
            jax.ShapeDtypeStruct((NSP, 16), jnp.float32),
        ],
        mesh=mesh,
        compiler_params=pltpu.CompilerParams(
            use_tc_tiling_on_sc=False, needs_layout_passes=False),
        scratch_types=[
            pltpu.VMEM((CH,), jnp.int32),
            pltpu.VMEM((CH,), jnp.int32),
            pltpu.VMEM((CH, C), jnp.float32),
            pltpu.VMEM((CH, C), jnp.float32),
            pltpu.VMEM((CH, 16), jnp.float32),
            pltpu.VMEM_SHARED((NSP, 16), jnp.float32),
            pltpu.SemaphoreType.DMA,
            pltpu.SemaphoreType.DMA,
        ],
    )
    return fn(q_dst, krel_src, s_pad, d_pad)


def _edge_phase(q_dst, krel_src, vrel_src, s, d, n_dst):
    pad = jnp.arange(E_PAD - E_REAL, dtype=jnp.int32) % N_NODE
    s_pad = jnp.concatenate([s.astype(jnp.int32), pad])
    d_pad = jnp.concatenate([d.astype(jnp.int32), pad])
    ex, den0, den1 = _sc_pass1(q_dst, krel_src, s_pad, d_pad)
    denom = (den0 + den1)[:n_dst, :H]
    a = ex[:E_REAL, :H] / (jnp.take(denom, d, axis=0) + 1e-16)
    ve = jnp.take(vrel_src, s, axis=0).reshape(-1, H, DH)
    msg = (ve * a[:, :, None]).reshape(-1, C)
    return jax.ops.segment_sum(msg, d, num_segments=n_dst)


_SRC_EDGE = {'user': 'ui', 'item': 'iu'}
_EDGE_DEFS = (('ui', 'user', 'item'), ('iu', 'item', 'user'))


def _fold_params(params):
    folded = {}
    inv_sqrt = 1.0 / math.sqrt(float(DH))
    for l in range(2):
        for t in ('user', 'item'):
            e = _SRC_EDGE[t]
            arel = params['l%d_arel_%s' % (l, e)]
            mrel = params['l%d_mrel_%s' % (l, e)]
            prel = params['l%d_prel_%s' % (l, e)] * inv_sqrt
            Wk = params['l%d_Wk_%s' % (l, t)].reshape(C, H, DH)
            Wv = params['l%d_Wv_%s' % (l, t)].reshape(C, H, DH)
            Wk_f = jnp.einsum('chd,hde,h->che', Wk, arel, prel).reshape(C, C)
            Wv_f = jnp.einsum('chd,hde->che', Wv, mrel).reshape(C, C)
            Wq = params['l%d_Wq_%s' % (l, t)]
            folded['Wqkv_%d_%s' % (l, t)] = jnp.concatenate([Wq, Wk_f, Wv_f], axis=1)
    return folded


def kernel(x_user, x_item, edge_index_user_item, edge_index_item_user, params):
    folded = _fold_params(params)
    h = {'user': _proj_relu(x_user, params['in_W_user'], params['in_b_user']),
         'item': _proj_relu(x_item, params['in_W_item'], params['in_b_item'])}
    ei = {'ui': (edge_index_user_item[0], edge_index_user_item[1]),
          'iu': (edge_index_item_user[0], edge_index_item_user[1])}
    for l in range(2):
        q, krel, vrel = {}, {}, {}
        for t in h:
            y = _proj(h[t], folded['Wqkv_%d_%s' % (l, t)])
            q[t] = y[:, :C]
            krel[t] = y[:, C:2 * C]
            vrel[t] = y[:, 2 * C:]
        out = {}
        for e, src, dst in _EDGE_DEFS:
            s, d = ei[e]
            out[dst] = _edge_phase(q[dst], krel[src], vrel[src], s, d, N_NODE)
        h_new = {}
        for t in h:
            beta = jax.nn.sigmoid(params['l%d_skip_%s' % (l, t)])
            hcoef = (1.0 - beta) + (1.0 if l > 0 else 0.0)
            h_new[t] = _out_stage(out[t], h[t],
                                  params['l%d_Wa_%s' % (l, t)],
                                  params['l%d_ba_%s' % (l, t)], beta, hcoef)
        h = h_new
    return (h['user'], h['item'])
